# baseline (speedup 1.0000x reference)
import functools
import jax
import jax.numpy as jnp
import numpy as np
from jax.experimental import pallas as pl
from jax.experimental.pallas import tpu as pltpu

_NUM_CLASSES = 13
_BN_SCALE = 1.0 / np.sqrt(1.0 + 1e-5)


def _square_distance(src, dst):
    return (jnp.sum(src ** 2, -1)[:, :, None] + jnp.sum(dst ** 2, -1)[:, None, :]
            - 2.0 * jnp.einsum('bnc,bmc->bnm', src, dst))


def _index_points(points, idx):
    return jax.vmap(lambda p, i: p[i])(points, idx)


def _fps(xyz, npoint):
    B, N, _ = xyz.shape

    def step(state, _):
        distance, farthest = state
        centroid = jnp.take_along_axis(
            xyz, jnp.broadcast_to(farthest[:, None, None], (B, 1, 3)), axis=1)
        dist = jnp.sum((xyz - centroid) ** 2, axis=-1)
        distance = jnp.minimum(distance, dist)
        new_far = jnp.argmax(distance, axis=-1).astype(jnp.int32)
        return (distance, new_far), farthest

    init = (jnp.full((B, N), 1e10, dtype=xyz.dtype), jnp.zeros((B,), dtype=jnp.int32))
    _, idxs = jax.lax.scan(step, init, None, length=npoint)
    return jnp.transpose(idxs, (1, 0))


def _ball_query(radius, nsample, xyz, new_xyz):
    B, S, _ = new_xyz.shape
    N = xyz.shape[1]
    sqr = _square_distance(new_xyz, xyz)
    grp = jnp.broadcast_to(jnp.arange(N, dtype=jnp.int32), (B, S, N))
    grp = jnp.where(sqr > radius * radius, N, grp)
    grp = jnp.sort(grp, axis=-1)[:, :, :nsample]
    first = grp[:, :, 0:1]
    grp = jnp.where(grp == N, first, grp)
    return grp


def _run_mlp(x, params, prefix, nlayers):
    for j in range(nlayers):
        w = params[prefix + '_w' + str(j)]
        g = params[prefix + '_g' + str(j)]
        b = params[prefix + '_b' + str(j)]
        x = jnp.einsum('...i,oi->...o', x, w)
        x = x * (g * _BN_SCALE) + b
        x = jax.nn.relu(x)
    return x


def _sa_module(xyz, points, params, prefix, npoint, radius, nsample):
    fidx = _fps_pallas(xyz, npoint)
    new_xyz = _index_points(xyz, fidx)
    sqr = _square_distance(new_xyz, xyz)
    gidx = _ball_query_pallas(radius, nsample, xyz, new_xyz, sqr)
    gs1 = params[prefix + '_g0'] * _BN_SCALE
    w1xt = (params[prefix + '_w0'][:, :3] * gs1[:, None]).T
    w1ft = (params[prefix + '_w0'][:, 3:] * gs1[:, None]).T
    table = _sa_table(xyz, points, w1xt, w1ft)
    B, S = gidx.shape[:2]
    gathered = _index_points(table, gidx.reshape(B, S * nsample))
    pooled = _sa_features(gathered, new_xyz, params, prefix, nsample)
    return new_xyz, pooled


def _fp_table_body(p_ref, w_ref, out_ref):
    out_ref[0] = jnp.dot(p_ref[0], w_ref[...], preferred_element_type=jnp.float32)


def _fp_table(points2, wbt):
    B, M, C = points2.shape
    C1 = wbt.shape[1]
    return pl.pallas_call(
        _fp_table_body,
        grid=(B,),
        in_specs=[
            pl.BlockSpec((1, M, C), lambda b: (b, 0, 0)),
            pl.BlockSpec(wbt.shape, lambda b: (0, 0)),
        ],
        out_specs=pl.BlockSpec((1, M, C1), lambda b: (b, 0, 0)),
        out_shape=jax.ShapeDtypeStruct((B, M, C1), jnp.float32),
    )(points2, wbt)


def _fp_mlp_body(p1_ref, tg_ref, w_ref, wat_ref, b1_ref, *rest, Sb, nlayers):
    layer_refs, out_ref = rest[:-1], rest[-1]
    tg = tg_ref[0]
    C1 = tg.shape[1]
    tg3 = tg.reshape(Sb, 3, C1)
    wv = w_ref[0]
    interp = (tg3[:, 0, :] * wv[:, 0:1] + tg3[:, 1, :] * wv[:, 1:2]) \
        + tg3[:, 2, :] * wv[:, 2:3]
    x = jnp.dot(p1_ref[0], wat_ref[...], preferred_element_type=jnp.float32)
    x = jax.nn.relu(x + interp + b1_ref[...])
    for j in range(nlayers - 1):
        wt, a, b = layer_refs[3 * j:3 * j + 3]
        x = jnp.dot(x, wt[...], preferred_element_type=jnp.float32)
        x = jax.nn.relu(x * a[...] + b[...])
    out_ref[0] = x


def _fp_module(xyz1, xyz2, points1, points2, params, prefix, nlayers):
    d = _square_distance(xyz1, xyz2)
    idx, w = _knn3_pallas(xyz1, xyz2, d)
    B, S, _ = xyz1.shape
    Cp = points1.shape[2]
    gs1 = params[prefix + '_g0'] * _BN_SCALE
    wat = (params[prefix + '_w0'][:, :Cp] * gs1[:, None]).T
    wbt = (params[prefix + '_w0'][:, Cp:] * gs1[:, None]).T
    table = _fp_table(points2, wbt)
    tg = _index_points(table, idx.reshape(B, S * 3))
    b1 = params[prefix + '_b0'][None, :]
    extras = []
    for j in range(1, nlayers):
        extras.append(params[prefix + '_w' + str(j)].T)
        extras.append((params[prefix + '_g' + str(j)] * _BN_SCALE)[None, :])
        extras.append(params[prefix + '_b' + str(j)][None, :])
    C1 = wat.shape[1]
    Cout = extras[-3].shape[1]
    Sb = min(S, 512)
    wspec = lambda a: pl.BlockSpec(a.shape, lambda bb, s: tuple(0 for _ in a.shape))
    out = pl.pallas_call(
        functools.partial(_fp_mlp_body, Sb=Sb, nlayers=nlayers),
        grid=(B, S // Sb),
        in_specs=[
            pl.BlockSpec((1, Sb, Cp), lambda b, s: (b, s, 0)),
            pl.BlockSpec((1, Sb * 3, C1), lambda b, s: (b, s, 0)),
            pl.BlockSpec((1, Sb, 3), lambda b, s: (b, s, 0)),
            wspec(wat), wspec(b1),
        ] + [wspec(a) for a in extras],
        out_specs=pl.BlockSpec((1, Sb, Cout), lambda b, s: (b, s, 0)),
        out_shape=jax.ShapeDtypeStruct((B, S, Cout), jnp.float32),
    )(points1, tg, w, wat, b1, *extras)
    return out


def _fps_body(xyz_ref, out_ref, *, npoint, N, B):
    R = N // 8
    PR = npoint // 8
    lin = (jax.lax.broadcasted_iota(jnp.int32, (8, R), 0) * R
           + jax.lax.broadcasted_iota(jnp.int32, (8, R), 1))
    plin = (jax.lax.broadcasted_iota(jnp.int32, (8, PR), 0) * PR
            + jax.lax.broadcasted_iota(jnp.int32, (8, PR), 1))
    chans = [[xyz_ref[b, c] for c in range(3)] for b in range(B)]

    def step(i, carry):
        new = []
        for b in range(B):
            d, far, acc = carry[b]
            acc = jnp.where(plin == i, far, acc)
            msk = lin == far
            x, y, z = chans[b]
            cx = jnp.sum(jnp.where(msk, x, 0.0), keepdims=True)
            cy = jnp.sum(jnp.where(msk, y, 0.0), keepdims=True)
            cz = jnp.sum(jnp.where(msk, z, 0.0), keepdims=True)
            dist = (x - cx) * (x - cx) + (y - cy) * (y - cy)
            dist = dist + (z - cz) * (z - cz)
            d = jnp.minimum(d, dist)
            m = jnp.max(d, keepdims=True)
            far = jnp.min(jnp.where(d == m, lin, N), keepdims=True)
            new.append((d, far, acc))
        return tuple(new)

    init = tuple((jnp.full((8, R), 1e10, jnp.float32),
                  jnp.zeros((1, 1), jnp.int32),
                  jnp.zeros((8, PR), jnp.int32)) for _ in range(B))
    final = jax.lax.fori_loop(0, npoint, step, init)
    for b in range(B):
        out_ref[b] = final[b][2]


def _fps_pallas(xyz, npoint):
    B, N, _ = xyz.shape
    xt = jnp.transpose(xyz, (0, 2, 1)).reshape(B, 3, 8, N // 8)
    out = pl.pallas_call(
        functools.partial(_fps_body, npoint=npoint, N=N, B=B),
        out_shape=jax.ShapeDtypeStruct((B, 8, npoint // 8), jnp.int32),
    )(xt)
    return out.reshape(B, npoint)


@functools.lru_cache(maxsize=None)
def _bq_consts(N, C, K):
    W = C // 16
    i = np.arange(N)
    k = np.arange(K)
    epool = (i[:, None] // C == k[None, :]).astype(np.float32)
    ecum = (i[:, None] < k[None, :] * C).astype(np.float32)
    pack = np.zeros((N, K * W), np.float32)
    pack[i, (i // C) * W + (i % C) // 16] = 2.0 ** (i % 16)
    return np.concatenate([epool, ecum, pack], axis=1)


def _bq_body(nx_ref, xt_ref, const_ref, out_ref, *, N, C, K, ns, r2):
    nx = nx_ref[0]
    xt = xt_ref[0]
    rn = jnp.sum(nx * nx, axis=1, keepdims=True)
    rm = (xt[0:1] * xt[0:1] + xt[1:2] * xt[1:2]) + xt[2:3] * xt[2:3]
    ab = jnp.dot(nx, xt, preferred_element_type=jnp.float32,
                 precision=jax.lax.Precision.HIGHEST)
    sqr = (rn + rm) - 2.0 * ab
    out_ref[0] = _bq_select(sqr, const_ref, C=C, K=K, ns=ns, r2=r2)


def _bq_body_pre(sqr_ref, const_ref, out_ref, *, N, C, K, ns, r2):
    out_ref[0] = _bq_select(sqr_ref[0], const_ref, C=C, K=K, ns=ns, r2=r2)


def _bq_select(sqr, const_ref, *, C, K, ns, r2):
    W = C // 16
    maskf = (sqr <= r2).astype(jnp.float32)
    agg = jnp.dot(maskf, const_ref[...], preferred_element_type=jnp.float32)
    cnt = agg[:, :K]
    base = agg[:, K:2 * K]
    wp = agg[:, 2 * K:]
    csum = base + cnt

    jrow = jax.lax.broadcasted_iota(jnp.int32, (1, ns), 1).astype(jnp.float32)
    chunk = jnp.zeros_like(jrow) + jnp.zeros((maskf.shape[0], 1), jnp.float32)
    for k in range(K):
        chunk = chunk + (csum[:, k:k + 1] <= jrow).astype(jnp.float32)
    base_sel = jnp.zeros_like(chunk)
    words = [jnp.zeros_like(chunk) for _ in range(W)]
    for k in range(K):
        sel = (chunk == k).astype(jnp.float32)
        base_sel = base_sel + base[:, k:k + 1] * sel
        for w in range(W):
            words[w] = words[w] + wp[:, k * W + w:k * W + w + 1] * sel
    r = jrow - base_sel
    run = jnp.zeros_like(chunk)
    loc = jnp.zeros_like(chunk)
    for w in range(W):
        word = words[w]
        for t in range(16):
            lo = jnp.floor(word * (1.0 / (1 << t)))
            hi = jnp.floor(word * (1.0 / (1 << (t + 1))))
            bit = lo - 2.0 * hi
            hit = (bit > 0.5) & (run == r)
            loc = loc + jnp.where(hit, float(w * 16 + t), 0.0)
            run = run + bit
    out = chunk * C + loc
    total = csum[:, K - 1:K]
    out = jnp.where(jrow < total, out, out[:, 0:1])
    return out.astype(jnp.int32)


def _ball_query_pallas(radius, nsample, xyz, new_xyz, sqr=None):
    B, S, _ = new_xyz.shape
    N = xyz.shape[1]
    C = min(N, 128)
    K = N // C
    Sb = min(S, 256)
    const = jnp.asarray(_bq_consts(N, C, K))
    r2 = np.float32(radius * radius)
    if sqr is None:
        body = functools.partial(_bq_body, N=N, C=C, K=K, ns=nsample, r2=r2)
        xt = jnp.transpose(xyz, (0, 2, 1))
        specs = [
            pl.BlockSpec((1, Sb, 3), lambda b, s: (b, s, 0)),
            pl.BlockSpec((1, 3, N), lambda b, s: (b, 0, 0)),
        ]
        args = (new_xyz, xt)
    else:
        body = functools.partial(_bq_body_pre, N=N, C=C, K=K, ns=nsample, r2=r2)
        specs = [pl.BlockSpec((1, Sb, N), lambda b, s: (b, s, 0))]
        args = (sqr,)
    out = pl.pallas_call(
        body,
        grid=(B, S // Sb),
        in_specs=specs + [pl.BlockSpec(const.shape, lambda b, s: (0, 0))],
        out_specs=pl.BlockSpec((1, Sb, nsample), lambda b, s: (b, s, 0)),
        out_shape=jax.ShapeDtypeStruct((B, S, nsample), jnp.int32),
    )(*args, const)
    return out


def _knn3_body(x1_ref, x2t_ref, idx_ref, w_ref, *, M):
    x1 = x1_ref[0]
    x2t = x2t_ref[0]
    rn = jnp.sum(x1 * x1, axis=1, keepdims=True)
    rm = (x2t[0:1] * x2t[0:1] + x2t[1:2] * x2t[1:2]) + x2t[2:3] * x2t[2:3]
    ab = jnp.dot(x1, x2t, preferred_element_type=jnp.float32,
                 precision=jax.lax.Precision.HIGHEST)
    d = (rn + rm) - 2.0 * ab
    _knn3_sel(d, idx_ref, w_ref, M)


def _knn3_body_pre(d_ref, idx_ref, w_ref, *, M):
    _knn3_sel(d_ref[0], idx_ref, w_ref, M)


def _knn3_sel(d, idx_ref, w_ref, M):
    iota = jax.lax.broadcasted_iota(jnp.int32, d.shape, 1)
    idxs, vals = [], []
    for _ in range(3):
        m = jnp.min(d, axis=1, keepdims=True)
        i = jnp.min(jnp.where(d == m, iota, M), axis=1, keepdims=True)
        d = jnp.where(iota == i, 1e30, d)
        idxs.append(i)
        vals.append(m)
    w = [1.0 / (jnp.maximum(v, 0.0) + 1e-8) for v in vals]
    ws = (w[0] + w[1]) + w[2]
    idx_ref[0] = jnp.concatenate(idxs, axis=1)
    w_ref[0] = jnp.concatenate([wj / ws for wj in w], axis=1)


def _knn3_pallas(xyz1, xyz2, d=None):
    B, S, _ = xyz1.shape
    M = xyz2.shape[1]
    Sb = min(S, 512)
    if d is None:
        body = functools.partial(_knn3_body, M=M)
        specs = [
            pl.BlockSpec((1, Sb, 3), lambda b, s: (b, s, 0)),
            pl.BlockSpec((1, 3, M), lambda b, s: (b, 0, 0)),
        ]
        args = (xyz1, jnp.transpose(xyz2, (0, 2, 1)))
    else:
        body = functools.partial(_knn3_body_pre, M=M)
        specs = [pl.BlockSpec((1, Sb, M), lambda b, s: (b, s, 0))]
        args = (d,)
    out = pl.pallas_call(
        body,
        grid=(B, S // Sb),
        in_specs=specs,
        out_specs=[
            pl.BlockSpec((1, Sb, 3), lambda b, s: (b, s, 0)),
            pl.BlockSpec((1, Sb, 3), lambda b, s: (b, s, 0)),
        ],
        out_shape=[
            jax.ShapeDtypeStruct((B, S, 3), jnp.int32),
            jax.ShapeDtypeStruct((B, S, 3), jnp.float32),
        ],
    )(*args)
    return out


def _sa_table_body(xyz_ref, feat_ref, w1xt_ref, w1ft_ref, out_ref):
    a = jnp.dot(xyz_ref[0], w1xt_ref[...], preferred_element_type=jnp.float32)
    a = a + jnp.dot(feat_ref[0], w1ft_ref[...], preferred_element_type=jnp.float32)
    out_ref[0] = a


def _sa_table(xyz, feat, w1xt, w1ft):
    B, N, _ = xyz.shape
    C1 = w1xt.shape[1]
    return pl.pallas_call(
        _sa_table_body,
        grid=(B,),
        in_specs=[
            pl.BlockSpec((1, N, 3), lambda b: (b, 0, 0)),
            pl.BlockSpec((1, N, feat.shape[2]), lambda b: (b, 0, 0)),
            pl.BlockSpec(w1xt.shape, lambda b: (0, 0)),
            pl.BlockSpec(w1ft.shape, lambda b: (0, 0)),
        ],
        out_specs=pl.BlockSpec((1, N, C1), lambda b: (b, 0, 0)),
        out_shape=jax.ShapeDtypeStruct((B, N, C1), jnp.float32),
    )(xyz, feat, w1xt, w1ft)


def _sa_mlp_body(g_ref, nx_ref, w1xt_ref, b1_ref, w2t_ref, a2_ref, b2_ref,
                 w3t_ref, a3_ref, b3_ref, out_ref, *, Sb, ns):
    gth = g_ref[0]
    off = jnp.dot(nx_ref[0], w1xt_ref[...],
                  preferred_element_type=jnp.float32) - b1_ref[...]
    C1 = gth.shape[1]
    off3 = jnp.broadcast_to(off[:, None, :], (Sb, ns, C1)).reshape(Sb * ns, C1)
    x1 = jax.nn.relu(gth - off3)
    x2 = jnp.dot(x1, w2t_ref[...], preferred_element_type=jnp.float32)
    x2 = jax.nn.relu(x2 * a2_ref[...] + b2_ref[...])
    x3 = jnp.dot(x2, w3t_ref[...], preferred_element_type=jnp.float32)
    x3 = jax.nn.relu(x3 * a3_ref[...] + b3_ref[...])
    C3 = x3.shape[1]
    out_ref[0] = jnp.max(x3.reshape(Sb, ns, C3), axis=1)


def _sa_features(gathered, new_xyz, params, prefix, ns):
    B, S, _ = new_xyz.shape
    gs1 = params[prefix + '_g0'] * _BN_SCALE
    w1xt = (params[prefix + '_w0'][:, :3] * gs1[:, None]).T
    b1 = params[prefix + '_b0'][None, :]
    w2t = params[prefix + '_w1'].T
    a2 = (params[prefix + '_g1'] * _BN_SCALE)[None, :]
    b2 = params[prefix + '_b1'][None, :]
    w3t = params[prefix + '_w2'].T
    a3 = (params[prefix + '_g2'] * _BN_SCALE)[None, :]
    b3 = params[prefix + '_b2'][None, :]
    C1, C3 = w2t.shape[0], w3t.shape[1]
    Sb = min(S, 256)
    wspec = lambda a: pl.BlockSpec(a.shape, lambda b, s: tuple(0 for _ in a.shape))
    out = pl.pallas_call(
        functools.partial(_sa_mlp_body, Sb=Sb, ns=ns),
        grid=(B, S // Sb),
        in_specs=[
            pl.BlockSpec((1, Sb * ns, C1), lambda b, s: (b, s, 0)),
            pl.BlockSpec((1, Sb, 3), lambda b, s: (b, s, 0)),
            wspec(w1xt), wspec(b1), wspec(w2t), wspec(a2), wspec(b2),
            wspec(w3t), wspec(a3), wspec(b3),
        ],
        out_specs=pl.BlockSpec((1, Sb, C3), lambda b, s: (b, s, 0)),
        out_shape=jax.ShapeDtypeStruct((B, S, C3), jnp.float32),
    )(gathered, new_xyz, w1xt, b1, w2t, a2, b2, w3t, a3, b3)
    return out


def _fc_head_body(x_ref, w1_ref, a1_ref, b1_ref, w2_ref, b2_ref, out_ref):
    x = x_ref[...]
    h = jnp.dot(x, w1_ref[...].T, preferred_element_type=jnp.float32)
    h = jax.nn.relu(h * a1_ref[...] + b1_ref[...])
    out_ref[...] = jnp.dot(h, w2_ref[...].T, preferred_element_type=jnp.float32) + b2_ref[...]


def _fc_head(x, w1, g1, b1, w2, b2):
    R, C = x.shape
    w2p = jnp.zeros((128, C), jnp.float32).at[:_NUM_CLASSES].set(w2)
    b2p = jnp.zeros((1, 128), jnp.float32).at[0, :_NUM_CLASSES].set(b2)
    a1 = (g1 * _BN_SCALE)[None, :]
    blk = 2048
    out = pl.pallas_call(
        _fc_head_body,
        grid=(R // blk,),
        in_specs=[
            pl.BlockSpec((blk, C), lambda i: (i, 0)),
            pl.BlockSpec((C, C), lambda i: (0, 0)),
            pl.BlockSpec((1, C), lambda i: (0, 0)),
            pl.BlockSpec((1, C), lambda i: (0, 0)),
            pl.BlockSpec((C, C), lambda i: (0, 0)),
            pl.BlockSpec((1, C), lambda i: (0, 0)),
        ],
        out_specs=pl.BlockSpec((blk, 128), lambda i: (i, 0)),
        out_shape=jax.ShapeDtypeStruct((R, 128), jnp.float32),
    )(x, w1, a1, b1[None, :], w2p, b2p)
    return out[:, :_NUM_CLASSES]


def kernel(xyz, points, params):
    l1x, l1p = _sa_module(xyz, points, params, 'sa0', 1024, 0.1, 32)
    l2x, l2p = _sa_module(l1x, l1p, params, 'sa1', 256, 0.2, 32)
    l3x, l3p = _sa_module(l2x, l2p, params, 'sa2', 64, 0.4, 32)
    l4x, l4p = _sa_module(l3x, l3p, params, 'sa3', 16, 0.8, 32)
    l3p = _fp_module(l3x, l4x, l3p, l4p, params, 'fp0', 2)
    l2p = _fp_module(l2x, l3x, l2p, l3p, params, 'fp1', 2)
    l1p = _fp_module(l1x, l2x, l1p, l2p, params, 'fp2', 2)
    l0p = _fp_module(xyz, l1x, points, l1p, params, 'fp3', 3)
    B, N, C = l0p.shape
    out = _fc_head(l0p.reshape(B * N, C), params['fc1_w'], params['fc1_g'],
                   params['fc1_b'], params['fc2_w'], params['fc2_bias'])
    return out.reshape(B, N, _NUM_CLASSES)


# speedup vs baseline: 1.0189x; 1.0189x over previous
import functools
import jax
import jax.numpy as jnp
import numpy as np
from jax.experimental import pallas as pl
from jax.experimental.pallas import tpu as pltpu

_NUM_CLASSES = 13
_BN_SCALE = 1.0 / np.sqrt(1.0 + 1e-5)


def _square_distance(src, dst):
    return (jnp.sum(src ** 2, -1)[:, :, None] + jnp.sum(dst ** 2, -1)[:, None, :]
            - 2.0 * jnp.einsum('bnc,bmc->bnm', src, dst))


def _index_points(points, idx):
    return jax.vmap(lambda p, i: p[i])(points, idx)


def _fps(xyz, npoint):
    B, N, _ = xyz.shape

    def step(state, _):
        distance, farthest = state
        centroid = jnp.take_along_axis(
            xyz, jnp.broadcast_to(farthest[:, None, None], (B, 1, 3)), axis=1)
        dist = jnp.sum((xyz - centroid) ** 2, axis=-1)
        distance = jnp.minimum(distance, dist)
        new_far = jnp.argmax(distance, axis=-1).astype(jnp.int32)
        return (distance, new_far), farthest

    init = (jnp.full((B, N), 1e10, dtype=xyz.dtype), jnp.zeros((B,), dtype=jnp.int32))
    _, idxs = jax.lax.scan(step, init, None, length=npoint)
    return jnp.transpose(idxs, (1, 0))


def _ball_query(radius, nsample, xyz, new_xyz):
    B, S, _ = new_xyz.shape
    N = xyz.shape[1]
    sqr = _square_distance(new_xyz, xyz)
    grp = jnp.broadcast_to(jnp.arange(N, dtype=jnp.int32), (B, S, N))
    grp = jnp.where(sqr > radius * radius, N, grp)
    grp = jnp.sort(grp, axis=-1)[:, :, :nsample]
    first = grp[:, :, 0:1]
    grp = jnp.where(grp == N, first, grp)
    return grp


def _run_mlp(x, params, prefix, nlayers):
    for j in range(nlayers):
        w = params[prefix + '_w' + str(j)]
        g = params[prefix + '_g' + str(j)]
        b = params[prefix + '_b' + str(j)]
        x = jnp.einsum('...i,oi->...o', x, w)
        x = x * (g * _BN_SCALE) + b
        x = jax.nn.relu(x)
    return x


def _sa_module(xyz, points, params, prefix, npoint, radius, nsample):
    fidx = _fps_pallas(xyz, npoint)
    new_xyz = _index_points(xyz, fidx)
    sqr = None if prefix == 'sa0' else _square_distance(new_xyz, xyz)
    gidx = _ball_query_pallas(radius, nsample, xyz, new_xyz, sqr)
    gs1 = params[prefix + '_g0'] * _BN_SCALE
    w1xt = (params[prefix + '_w0'][:, :3] * gs1[:, None]).T
    w1ft = (params[prefix + '_w0'][:, 3:] * gs1[:, None]).T
    table = _sa_table(xyz, points, w1xt, w1ft)
    B, S = gidx.shape[:2]
    gathered = _index_points(table, gidx.reshape(B, S * nsample))
    pooled = _sa_features(gathered, new_xyz, params, prefix, nsample)
    return new_xyz, pooled


def _fp_table_body(p_ref, w_ref, out_ref):
    out_ref[0] = jnp.dot(p_ref[0], w_ref[...], preferred_element_type=jnp.float32)


def _fp_table(points2, wbt):
    B, M, C = points2.shape
    C1 = wbt.shape[1]
    return pl.pallas_call(
        _fp_table_body,
        grid=(B,),
        in_specs=[
            pl.BlockSpec((1, M, C), lambda b: (b, 0, 0)),
            pl.BlockSpec(wbt.shape, lambda b: (0, 0)),
        ],
        out_specs=pl.BlockSpec((1, M, C1), lambda b: (b, 0, 0)),
        out_shape=jax.ShapeDtypeStruct((B, M, C1), jnp.float32),
    )(points2, wbt)


def _fp_mlp_body(p1_ref, tg_ref, w_ref, wat_ref, b1_ref, *rest, Sb, nlayers):
    layer_refs, out_ref = rest[:-1], rest[-1]
    tg = tg_ref[0]
    C1 = tg.shape[1]
    tg3 = tg.reshape(Sb, 3, C1)
    wv = w_ref[0]
    interp = (tg3[:, 0, :] * wv[:, 0:1] + tg3[:, 1, :] * wv[:, 1:2]) \
        + tg3[:, 2, :] * wv[:, 2:3]
    x = jnp.dot(p1_ref[0], wat_ref[...], preferred_element_type=jnp.float32)
    x = jax.nn.relu(x + interp + b1_ref[...])
    for j in range(nlayers - 1):
        wt, a, b = layer_refs[3 * j:3 * j + 3]
        x = jnp.dot(x, wt[...], preferred_element_type=jnp.float32)
        x = jax.nn.relu(x * a[...] + b[...])
    out_ref[0] = x


def _fp_module(xyz1, xyz2, points1, points2, params, prefix, nlayers):
    d = None if prefix == 'fp3' else _square_distance(xyz1, xyz2)
    idx, w = _knn3_pallas(xyz1, xyz2, d)
    B, S, _ = xyz1.shape
    Cp = points1.shape[2]
    gs1 = params[prefix + '_g0'] * _BN_SCALE
    wat = (params[prefix + '_w0'][:, :Cp] * gs1[:, None]).T
    wbt = (params[prefix + '_w0'][:, Cp:] * gs1[:, None]).T
    table = _fp_table(points2, wbt)
    tg = _index_points(table, idx.reshape(B, S * 3))
    b1 = params[prefix + '_b0'][None, :]
    extras = []
    for j in range(1, nlayers):
        extras.append(params[prefix + '_w' + str(j)].T)
        extras.append((params[prefix + '_g' + str(j)] * _BN_SCALE)[None, :])
        extras.append(params[prefix + '_b' + str(j)][None, :])
    C1 = wat.shape[1]
    Cout = extras[-3].shape[1]
    Sb = min(S, 512)
    wspec = lambda a: pl.BlockSpec(a.shape, lambda bb, s: tuple(0 for _ in a.shape))
    out = pl.pallas_call(
        functools.partial(_fp_mlp_body, Sb=Sb, nlayers=nlayers),
        grid=(B, S // Sb),
        in_specs=[
            pl.BlockSpec((1, Sb, Cp), lambda b, s: (b, s, 0)),
            pl.BlockSpec((1, Sb * 3, C1), lambda b, s: (b, s, 0)),
            pl.BlockSpec((1, Sb, 3), lambda b, s: (b, s, 0)),
            wspec(wat), wspec(b1),
        ] + [wspec(a) for a in extras],
        out_specs=pl.BlockSpec((1, Sb, Cout), lambda b, s: (b, s, 0)),
        out_shape=jax.ShapeDtypeStruct((B, S, Cout), jnp.float32),
    )(points1, tg, w, wat, b1, *extras)
    return out


def _fps_body(xyz_ref, out_ref, *, npoint, N, B):
    R = N // 8
    PR = npoint // 8
    lin = (jax.lax.broadcasted_iota(jnp.int32, (8, R), 0) * R
           + jax.lax.broadcasted_iota(jnp.int32, (8, R), 1))
    plin = (jax.lax.broadcasted_iota(jnp.int32, (8, PR), 0) * PR
            + jax.lax.broadcasted_iota(jnp.int32, (8, PR), 1))
    chans = [[xyz_ref[b, c] for c in range(3)] for b in range(B)]

    def step(i, carry):
        new = []
        for b in range(B):
            d, far, acc = carry[b]
            acc = jnp.where(plin == i, far, acc)
            msk = lin == far
            x, y, z = chans[b]
            cx = jnp.sum(jnp.where(msk, x, 0.0), keepdims=True)
            cy = jnp.sum(jnp.where(msk, y, 0.0), keepdims=True)
            cz = jnp.sum(jnp.where(msk, z, 0.0), keepdims=True)
            dist = (x - cx) * (x - cx) + (y - cy) * (y - cy)
            dist = dist + (z - cz) * (z - cz)
            d = jnp.minimum(d, dist)
            m = jnp.max(d, keepdims=True)
            far = jnp.min(jnp.where(d == m, lin, N), keepdims=True)
            new.append((d, far, acc))
        return tuple(new)

    init = tuple((jnp.full((8, R), 1e10, jnp.float32),
                  jnp.zeros((1, 1), jnp.int32),
                  jnp.zeros((8, PR), jnp.int32)) for _ in range(B))
    final = jax.lax.fori_loop(0, npoint, step, init)
    for b in range(B):
        out_ref[b] = final[b][2]


def _fps_pallas(xyz, npoint):
    B, N, _ = xyz.shape
    xt = jnp.transpose(xyz, (0, 2, 1)).reshape(B, 3, 8, N // 8)
    out = pl.pallas_call(
        functools.partial(_fps_body, npoint=npoint, N=N, B=B),
        out_shape=jax.ShapeDtypeStruct((B, 8, npoint // 8), jnp.int32),
    )(xt)
    return out.reshape(B, npoint)


@functools.lru_cache(maxsize=None)
def _bq_consts(N, C, K):
    W = C // 16
    i = np.arange(N)
    k = np.arange(K)
    epool = (i[:, None] // C == k[None, :]).astype(np.float32)
    ecum = (i[:, None] < k[None, :] * C).astype(np.float32)
    pack = np.zeros((N, K * W), np.float32)
    pack[i, (i // C) * W + (i % C) // 16] = 2.0 ** (i % 16)
    return np.concatenate([epool, ecum, pack], axis=1)


def _bq_body(nx_ref, xt_ref, const_ref, out_ref, *, N, C, K, ns, r2):
    nx = nx_ref[0]
    xt = xt_ref[0]
    rn = jnp.sum(nx * nx, axis=1, keepdims=True)
    rm = (xt[0:1] * xt[0:1] + xt[1:2] * xt[1:2]) + xt[2:3] * xt[2:3]
    ab = jnp.dot(nx, xt, preferred_element_type=jnp.float32)
    sqr = (rn + rm) - 2.0 * ab
    out_ref[0] = _bq_select(sqr, const_ref, C=C, K=K, ns=ns, r2=r2)


def _bq_body_pre(sqr_ref, const_ref, out_ref, *, N, C, K, ns, r2):
    out_ref[0] = _bq_select(sqr_ref[0], const_ref, C=C, K=K, ns=ns, r2=r2)


def _bq_select(sqr, const_ref, *, C, K, ns, r2):
    W = C // 16
    maskf = (sqr <= r2).astype(jnp.float32)
    agg = jnp.dot(maskf, const_ref[...], preferred_element_type=jnp.float32)
    cnt = agg[:, :K]
    base = agg[:, K:2 * K]
    wp = agg[:, 2 * K:]
    csum = base + cnt

    jrow = jax.lax.broadcasted_iota(jnp.int32, (1, ns), 1).astype(jnp.float32)
    chunk = jnp.zeros_like(jrow) + jnp.zeros((maskf.shape[0], 1), jnp.float32)
    for k in range(K):
        chunk = chunk + (csum[:, k:k + 1] <= jrow).astype(jnp.float32)
    base_sel = jnp.zeros_like(chunk)
    words = [jnp.zeros_like(chunk) for _ in range(W)]
    for k in range(K):
        sel = (chunk == k).astype(jnp.float32)
        base_sel = base_sel + base[:, k:k + 1] * sel
        for w in range(W):
            words[w] = words[w] + wp[:, k * W + w:k * W + w + 1] * sel
    r = jrow - base_sel
    run = jnp.zeros_like(chunk)
    loc = jnp.zeros_like(chunk)
    for w in range(W):
        word = words[w]
        for t in range(16):
            lo = jnp.floor(word * (1.0 / (1 << t)))
            hi = jnp.floor(word * (1.0 / (1 << (t + 1))))
            bit = lo - 2.0 * hi
            hit = (bit > 0.5) & (run == r)
            loc = loc + jnp.where(hit, float(w * 16 + t), 0.0)
            run = run + bit
    out = chunk * C + loc
    total = csum[:, K - 1:K]
    out = jnp.where(jrow < total, out, out[:, 0:1])
    return out.astype(jnp.int32)


def _ball_query_pallas(radius, nsample, xyz, new_xyz, sqr=None):
    B, S, _ = new_xyz.shape
    N = xyz.shape[1]
    C = min(N, 128)
    K = N // C
    Sb = min(S, 256)
    const = jnp.asarray(_bq_consts(N, C, K))
    r2 = np.float32(radius * radius)
    if sqr is None:
        body = functools.partial(_bq_body, N=N, C=C, K=K, ns=nsample, r2=r2)
        xt = jnp.transpose(xyz, (0, 2, 1))
        specs = [
            pl.BlockSpec((1, Sb, 3), lambda b, s: (b, s, 0)),
            pl.BlockSpec((1, 3, N), lambda b, s: (b, 0, 0)),
        ]
        args = (new_xyz, xt)
    else:
        body = functools.partial(_bq_body_pre, N=N, C=C, K=K, ns=nsample, r2=r2)
        specs = [pl.BlockSpec((1, Sb, N), lambda b, s: (b, s, 0))]
        args = (sqr,)
    out = pl.pallas_call(
        body,
        grid=(B, S // Sb),
        in_specs=specs + [pl.BlockSpec(const.shape, lambda b, s: (0, 0))],
        out_specs=pl.BlockSpec((1, Sb, nsample), lambda b, s: (b, s, 0)),
        out_shape=jax.ShapeDtypeStruct((B, S, nsample), jnp.int32),
    )(*args, const)
    return out


def _knn3_body(x1_ref, x2t_ref, idx_ref, w_ref, *, M):
    x1 = x1_ref[0]
    x2t = x2t_ref[0]
    rn = jnp.sum(x1 * x1, axis=1, keepdims=True)
    rm = (x2t[0:1] * x2t[0:1] + x2t[1:2] * x2t[1:2]) + x2t[2:3] * x2t[2:3]
    ab = jnp.dot(x1, x2t, preferred_element_type=jnp.float32)
    d = (rn + rm) - 2.0 * ab
    _knn3_sel(d, idx_ref, w_ref, M)


def _knn3_body_pre(d_ref, idx_ref, w_ref, *, M):
    _knn3_sel(d_ref[0], idx_ref, w_ref, M)


def _knn3_sel(d, idx_ref, w_ref, M):
    iota = jax.lax.broadcasted_iota(jnp.int32, d.shape, 1)
    idxs, vals = [], []
    for _ in range(3):
        m = jnp.min(d, axis=1, keepdims=True)
        i = jnp.min(jnp.where(d == m, iota, M), axis=1, keepdims=True)
        d = jnp.where(iota == i, 1e30, d)
        idxs.append(i)
        vals.append(m)
    w = [1.0 / (jnp.maximum(v, 0.0) + 1e-8) for v in vals]
    ws = (w[0] + w[1]) + w[2]
    idx_ref[0] = jnp.concatenate(idxs, axis=1)
    w_ref[0] = jnp.concatenate([wj / ws for wj in w], axis=1)


def _knn3_pallas(xyz1, xyz2, d=None):
    B, S, _ = xyz1.shape
    M = xyz2.shape[1]
    Sb = min(S, 512)
    if d is None:
        body = functools.partial(_knn3_body, M=M)
        specs = [
            pl.BlockSpec((1, Sb, 3), lambda b, s: (b, s, 0)),
            pl.BlockSpec((1, 3, M), lambda b, s: (b, 0, 0)),
        ]
        args = (xyz1, jnp.transpose(xyz2, (0, 2, 1)))
    else:
        body = functools.partial(_knn3_body_pre, M=M)
        specs = [pl.BlockSpec((1, Sb, M), lambda b, s: (b, s, 0))]
        args = (d,)
    out = pl.pallas_call(
        body,
        grid=(B, S // Sb),
        in_specs=specs,
        out_specs=[
            pl.BlockSpec((1, Sb, 3), lambda b, s: (b, s, 0)),
            pl.BlockSpec((1, Sb, 3), lambda b, s: (b, s, 0)),
        ],
        out_shape=[
            jax.ShapeDtypeStruct((B, S, 3), jnp.int32),
            jax.ShapeDtypeStruct((B, S, 3), jnp.float32),
        ],
    )(*args)
    return out


def _sa_table_body(xyz_ref, feat_ref, w1xt_ref, w1ft_ref, out_ref):
    a = jnp.dot(xyz_ref[0], w1xt_ref[...], preferred_element_type=jnp.float32)
    a = a + jnp.dot(feat_ref[0], w1ft_ref[...], preferred_element_type=jnp.float32)
    out_ref[0] = a


def _sa_table(xyz, feat, w1xt, w1ft):
    B, N, _ = xyz.shape
    C1 = w1xt.shape[1]
    return pl.pallas_call(
        _sa_table_body,
        grid=(B,),
        in_specs=[
            pl.BlockSpec((1, N, 3), lambda b: (b, 0, 0)),
            pl.BlockSpec((1, N, feat.shape[2]), lambda b: (b, 0, 0)),
            pl.BlockSpec(w1xt.shape, lambda b: (0, 0)),
            pl.BlockSpec(w1ft.shape, lambda b: (0, 0)),
        ],
        out_specs=pl.BlockSpec((1, N, C1), lambda b: (b, 0, 0)),
        out_shape=jax.ShapeDtypeStruct((B, N, C1), jnp.float32),
    )(xyz, feat, w1xt, w1ft)


def _sa_mlp_body(g_ref, nx_ref, w1xt_ref, b1_ref, w2t_ref, a2_ref, b2_ref,
                 w3t_ref, a3_ref, b3_ref, out_ref, *, Sb, ns):
    gth = g_ref[0]
    off = jnp.dot(nx_ref[0], w1xt_ref[...],
                  preferred_element_type=jnp.float32) - b1_ref[...]
    C1 = gth.shape[1]
    off3 = jnp.broadcast_to(off[:, None, :], (Sb, ns, C1)).reshape(Sb * ns, C1)
    x1 = jax.nn.relu(gth - off3)
    x2 = jnp.dot(x1, w2t_ref[...], preferred_element_type=jnp.float32)
    x2 = jax.nn.relu(x2 * a2_ref[...] + b2_ref[...])
    x3 = jnp.dot(x2, w3t_ref[...], preferred_element_type=jnp.float32)
    x3 = jax.nn.relu(x3 * a3_ref[...] + b3_ref[...])
    C3 = x3.shape[1]
    out_ref[0] = jnp.max(x3.reshape(Sb, ns, C3), axis=1)


def _sa_features(gathered, new_xyz, params, prefix, ns):
    B, S, _ = new_xyz.shape
    gs1 = params[prefix + '_g0'] * _BN_SCALE
    w1xt = (params[prefix + '_w0'][:, :3] * gs1[:, None]).T
    b1 = params[prefix + '_b0'][None, :]
    w2t = params[prefix + '_w1'].T
    a2 = (params[prefix + '_g1'] * _BN_SCALE)[None, :]
    b2 = params[prefix + '_b1'][None, :]
    w3t = params[prefix + '_w2'].T
    a3 = (params[prefix + '_g2'] * _BN_SCALE)[None, :]
    b3 = params[prefix + '_b2'][None, :]
    C1, C3 = w2t.shape[0], w3t.shape[1]
    Sb = min(S, 256)
    wspec = lambda a: pl.BlockSpec(a.shape, lambda b, s: tuple(0 for _ in a.shape))
    out = pl.pallas_call(
        functools.partial(_sa_mlp_body, Sb=Sb, ns=ns),
        grid=(B, S // Sb),
        in_specs=[
            pl.BlockSpec((1, Sb * ns, C1), lambda b, s: (b, s, 0)),
            pl.BlockSpec((1, Sb, 3), lambda b, s: (b, s, 0)),
            wspec(w1xt), wspec(b1), wspec(w2t), wspec(a2), wspec(b2),
            wspec(w3t), wspec(a3), wspec(b3),
        ],
        out_specs=pl.BlockSpec((1, Sb, C3), lambda b, s: (b, s, 0)),
        out_shape=jax.ShapeDtypeStruct((B, S, C3), jnp.float32),
    )(gathered, new_xyz, w1xt, b1, w2t, a2, b2, w3t, a3, b3)
    return out


def _fc_head_body(x_ref, w1_ref, a1_ref, b1_ref, w2_ref, b2_ref, out_ref):
    x = x_ref[...]
    h = jnp.dot(x, w1_ref[...].T, preferred_element_type=jnp.float32)
    h = jax.nn.relu(h * a1_ref[...] + b1_ref[...])
    out_ref[...] = jnp.dot(h, w2_ref[...].T, preferred_element_type=jnp.float32) + b2_ref[...]


def _fc_head(x, w1, g1, b1, w2, b2):
    R, C = x.shape
    w2p = jnp.zeros((128, C), jnp.float32).at[:_NUM_CLASSES].set(w2)
    b2p = jnp.zeros((1, 128), jnp.float32).at[0, :_NUM_CLASSES].set(b2)
    a1 = (g1 * _BN_SCALE)[None, :]
    blk = 2048
    out = pl.pallas_call(
        _fc_head_body,
        grid=(R // blk,),
        in_specs=[
            pl.BlockSpec((blk, C), lambda i: (i, 0)),
            pl.BlockSpec((C, C), lambda i: (0, 0)),
            pl.BlockSpec((1, C), lambda i: (0, 0)),
            pl.BlockSpec((1, C), lambda i: (0, 0)),
            pl.BlockSpec((C, C), lambda i: (0, 0)),
            pl.BlockSpec((1, C), lambda i: (0, 0)),
        ],
        out_specs=pl.BlockSpec((blk, 128), lambda i: (i, 0)),
        out_shape=jax.ShapeDtypeStruct((R, 128), jnp.float32),
    )(x, w1, a1, b1[None, :], w2p, b2p)
    return out[:, :_NUM_CLASSES]


def kernel(xyz, points, params):
    l1x, l1p = _sa_module(xyz, points, params, 'sa0', 1024, 0.1, 32)
    l2x, l2p = _sa_module(l1x, l1p, params, 'sa1', 256, 0.2, 32)
    l3x, l3p = _sa_module(l2x, l2p, params, 'sa2', 64, 0.4, 32)
    l4x, l4p = _sa_module(l3x, l3p, params, 'sa3', 16, 0.8, 32)
    l3p = _fp_module(l3x, l4x, l3p, l4p, params, 'fp0', 2)
    l2p = _fp_module(l2x, l3x, l2p, l3p, params, 'fp1', 2)
    l1p = _fp_module(l1x, l2x, l1p, l2p, params, 'fp2', 2)
    l0p = _fp_module(xyz, l1x, points, l1p, params, 'fp3', 3)
    B, N, C = l0p.shape
    out = _fc_head(l0p.reshape(B * N, C), params['fc1_w'], params['fc1_g'],
                   params['fc1_b'], params['fc2_w'], params['fc2_bias'])
    return out.reshape(B, N, _NUM_CLASSES)


# speedup vs baseline: 1.2269x; 1.2042x over previous
import functools
import jax
import jax.numpy as jnp
import numpy as np
from jax import lax
from jax.experimental import pallas as pl
from jax.experimental.pallas import tpu as pltpu
from jax.experimental.pallas import tpu_sc as plsc

_NUM_CLASSES = 13
_BN_SCALE = 1.0 / np.sqrt(1.0 + 1e-5)


def _square_distance(src, dst):
    return (jnp.sum(src ** 2, -1)[:, :, None] + jnp.sum(dst ** 2, -1)[:, None, :]
            - 2.0 * jnp.einsum('bnc,bmc->bnm', src, dst))


def _index_points(points, idx):
    return jax.vmap(lambda p, i: p[i])(points, idx)


def _fps(xyz, npoint):
    B, N, _ = xyz.shape

    def step(state, _):
        distance, farthest = state
        centroid = jnp.take_along_axis(
            xyz, jnp.broadcast_to(farthest[:, None, None], (B, 1, 3)), axis=1)
        dist = jnp.sum((xyz - centroid) ** 2, axis=-1)
        distance = jnp.minimum(distance, dist)
        new_far = jnp.argmax(distance, axis=-1).astype(jnp.int32)
        return (distance, new_far), farthest

    init = (jnp.full((B, N), 1e10, dtype=xyz.dtype), jnp.zeros((B,), dtype=jnp.int32))
    _, idxs = jax.lax.scan(step, init, None, length=npoint)
    return jnp.transpose(idxs, (1, 0))


def _ball_query(radius, nsample, xyz, new_xyz):
    B, S, _ = new_xyz.shape
    N = xyz.shape[1]
    sqr = _square_distance(new_xyz, xyz)
    grp = jnp.broadcast_to(jnp.arange(N, dtype=jnp.int32), (B, S, N))
    grp = jnp.where(sqr > radius * radius, N, grp)
    grp = jnp.sort(grp, axis=-1)[:, :, :nsample]
    first = grp[:, :, 0:1]
    grp = jnp.where(grp == N, first, grp)
    return grp


def _run_mlp(x, params, prefix, nlayers):
    for j in range(nlayers):
        w = params[prefix + '_w' + str(j)]
        g = params[prefix + '_g' + str(j)]
        b = params[prefix + '_b' + str(j)]
        x = jnp.einsum('...i,oi->...o', x, w)
        x = x * (g * _BN_SCALE) + b
        x = jax.nn.relu(x)
    return x


def _sa_module(xyz, points, params, prefix, npoint, radius, nsample):
    fidx = _fps_pallas(xyz, npoint)
    new_xyz = _index_points(xyz, fidx)
    sqr = None if prefix == 'sa0' else _square_distance(new_xyz, xyz)
    gidx = _ball_query_pallas(radius, nsample, xyz, new_xyz, sqr)
    gs1 = params[prefix + '_g0'] * _BN_SCALE
    w1xt = (params[prefix + '_w0'][:, :3] * gs1[:, None]).T
    w1ft = (params[prefix + '_w0'][:, 3:] * gs1[:, None]).T
    table = _sa_table(xyz, points, w1xt, w1ft)
    B, S = gidx.shape[:2]
    gathered = _gather_rows(table, gidx.reshape(B, S * nsample))
    pooled = _sa_features(gathered, new_xyz, params, prefix, nsample)
    return new_xyz, pooled


def _fp_table_body(p_ref, w_ref, out_ref):
    out_ref[0] = jnp.dot(p_ref[0], w_ref[...], preferred_element_type=jnp.float32)


def _fp_table(points2, wbt):
    B, M, C = points2.shape
    C1 = wbt.shape[1]
    return pl.pallas_call(
        _fp_table_body,
        grid=(B,),
        in_specs=[
            pl.BlockSpec((1, M, C), lambda b: (b, 0, 0)),
            pl.BlockSpec(wbt.shape, lambda b: (0, 0)),
        ],
        out_specs=pl.BlockSpec((1, M, C1), lambda b: (b, 0, 0)),
        out_shape=jax.ShapeDtypeStruct((B, M, C1), jnp.float32),
    )(points2, wbt)


def _fp_mlp_body(p1_ref, tg_ref, w_ref, wat_ref, b1_ref, *rest, Sb, nlayers):
    layer_refs, out_ref = rest[:-1], rest[-1]
    tg = tg_ref[0]
    C1 = tg.shape[1]
    tg3 = tg.reshape(Sb, 3, C1)
    wv = w_ref[0]
    interp = (tg3[:, 0, :] * wv[:, 0:1] + tg3[:, 1, :] * wv[:, 1:2]) \
        + tg3[:, 2, :] * wv[:, 2:3]
    x = jnp.dot(p1_ref[0], wat_ref[...], preferred_element_type=jnp.float32)
    x = jax.nn.relu(x + interp + b1_ref[...])
    for j in range(nlayers - 1):
        wt, a, b = layer_refs[3 * j:3 * j + 3]
        x = jnp.dot(x, wt[...], preferred_element_type=jnp.float32)
        x = jax.nn.relu(x * a[...] + b[...])
    out_ref[0] = x


def _fp_module(xyz1, xyz2, points1, points2, params, prefix, nlayers):
    d = None if prefix == 'fp3' else _square_distance(xyz1, xyz2)
    idx, w = _knn3_pallas(xyz1, xyz2, d)
    B, S, _ = xyz1.shape
    Cp = points1.shape[2]
    gs1 = params[prefix + '_g0'] * _BN_SCALE
    wat = (params[prefix + '_w0'][:, :Cp] * gs1[:, None]).T
    wbt = (params[prefix + '_w0'][:, Cp:] * gs1[:, None]).T
    table = _fp_table(points2, wbt)
    tg = _gather_rows(table, idx.reshape(B, S * 3))
    b1 = params[prefix + '_b0'][None, :]
    extras = []
    for j in range(1, nlayers):
        extras.append(params[prefix + '_w' + str(j)].T)
        extras.append((params[prefix + '_g' + str(j)] * _BN_SCALE)[None, :])
        extras.append(params[prefix + '_b' + str(j)][None, :])
    C1 = wat.shape[1]
    Cout = extras[-3].shape[1]
    Sb = min(S, 512)
    wspec = lambda a: pl.BlockSpec(a.shape, lambda bb, s: tuple(0 for _ in a.shape))
    out = pl.pallas_call(
        functools.partial(_fp_mlp_body, Sb=Sb, nlayers=nlayers),
        grid=(B, S // Sb),
        in_specs=[
            pl.BlockSpec((1, Sb, Cp), lambda b, s: (b, s, 0)),
            pl.BlockSpec((1, Sb * 3, C1), lambda b, s: (b, s, 0)),
            pl.BlockSpec((1, Sb, 3), lambda b, s: (b, s, 0)),
            wspec(wat), wspec(b1),
        ] + [wspec(a) for a in extras],
        out_specs=pl.BlockSpec((1, Sb, Cout), lambda b, s: (b, s, 0)),
        out_shape=jax.ShapeDtypeStruct((B, S, Cout), jnp.float32),
    )(points1, tg, w, wat, b1, *extras)
    return out


def _fps_body(xyz_ref, out_ref, *, npoint, N, B):
    R = N // 8
    PR = npoint // 8
    lin = (jax.lax.broadcasted_iota(jnp.int32, (8, R), 0) * R
           + jax.lax.broadcasted_iota(jnp.int32, (8, R), 1))
    plin = (jax.lax.broadcasted_iota(jnp.int32, (8, PR), 0) * PR
            + jax.lax.broadcasted_iota(jnp.int32, (8, PR), 1))
    chans = [[xyz_ref[b, c] for c in range(3)] for b in range(B)]

    def step(i, carry):
        new = []
        for b in range(B):
            d, far, acc = carry[b]
            acc = jnp.where(plin == i, far, acc)
            msk = lin == far
            x, y, z = chans[b]
            cx = jnp.sum(jnp.where(msk, x, 0.0), keepdims=True)
            cy = jnp.sum(jnp.where(msk, y, 0.0), keepdims=True)
            cz = jnp.sum(jnp.where(msk, z, 0.0), keepdims=True)
            dist = (x - cx) * (x - cx) + (y - cy) * (y - cy)
            dist = dist + (z - cz) * (z - cz)
            d = jnp.minimum(d, dist)
            m = jnp.max(d, keepdims=True)
            far = jnp.min(jnp.where(d == m, lin, N), keepdims=True)
            new.append((d, far, acc))
        return tuple(new)

    init = tuple((jnp.full((8, R), 1e10, jnp.float32),
                  jnp.zeros((1, 1), jnp.int32),
                  jnp.zeros((8, PR), jnp.int32)) for _ in range(B))
    final = jax.lax.fori_loop(0, npoint, step, init)
    for b in range(B):
        out_ref[b] = final[b][2]


def _fps_pallas(xyz, npoint):
    B, N, _ = xyz.shape
    xt = jnp.transpose(xyz, (0, 2, 1)).reshape(B, 3, 8, N // 8)
    out = pl.pallas_call(
        functools.partial(_fps_body, npoint=npoint, N=N, B=B),
        out_shape=jax.ShapeDtypeStruct((B, 8, npoint // 8), jnp.int32),
    )(xt)
    return out.reshape(B, npoint)


@functools.lru_cache(maxsize=None)
def _bq_consts(N, C, K):
    W = C // 16
    i = np.arange(N)
    k = np.arange(K)
    epool = (i[:, None] // C == k[None, :]).astype(np.float32)
    ecum = (i[:, None] < k[None, :] * C).astype(np.float32)
    pack = np.zeros((N, K * W), np.float32)
    pack[i, (i // C) * W + (i % C) // 16] = 2.0 ** (i % 16)
    return np.concatenate([epool, ecum, pack], axis=1)


def _bq_body(nx_ref, xt_ref, const_ref, out_ref, *, N, C, K, ns, r2):
    nx = nx_ref[0]
    xt = xt_ref[0]
    rn = jnp.sum(nx * nx, axis=1, keepdims=True)
    rm = (xt[0:1] * xt[0:1] + xt[1:2] * xt[1:2]) + xt[2:3] * xt[2:3]
    ab = jnp.dot(nx, xt, preferred_element_type=jnp.float32)
    sqr = (rn + rm) - 2.0 * ab
    out_ref[0] = _bq_select(sqr, const_ref, C=C, K=K, ns=ns, r2=r2)


def _bq_body_pre(sqr_ref, const_ref, out_ref, *, N, C, K, ns, r2):
    out_ref[0] = _bq_select(sqr_ref[0], const_ref, C=C, K=K, ns=ns, r2=r2)


def _bq_select(sqr, const_ref, *, C, K, ns, r2):
    W = C // 16
    maskf = (sqr <= r2).astype(jnp.float32)
    agg = jnp.dot(maskf, const_ref[...], preferred_element_type=jnp.float32)
    cnt = agg[:, :K]
    base = agg[:, K:2 * K]
    wp = agg[:, 2 * K:]
    csum = base + cnt

    jrow = jax.lax.broadcasted_iota(jnp.int32, (1, ns), 1).astype(jnp.float32)
    chunk = jnp.zeros_like(jrow) + jnp.zeros((maskf.shape[0], 1), jnp.float32)
    for k in range(K):
        chunk = chunk + (csum[:, k:k + 1] <= jrow).astype(jnp.float32)
    base_sel = jnp.zeros_like(chunk)
    words = [jnp.zeros_like(chunk) for _ in range(W)]
    for k in range(K):
        sel = (chunk == k).astype(jnp.float32)
        base_sel = base_sel + base[:, k:k + 1] * sel
        for w in range(W):
            words[w] = words[w] + wp[:, k * W + w:k * W + w + 1] * sel
    r = jrow - base_sel
    run = jnp.zeros_like(chunk)
    loc = jnp.zeros_like(chunk)
    for w in range(W):
        word = words[w]
        for t in range(16):
            lo = jnp.floor(word * (1.0 / (1 << t)))
            hi = jnp.floor(word * (1.0 / (1 << (t + 1))))
            bit = lo - 2.0 * hi
            hit = (bit > 0.5) & (run == r)
            loc = loc + jnp.where(hit, float(w * 16 + t), 0.0)
            run = run + bit
    out = chunk * C + loc
    total = csum[:, K - 1:K]
    out = jnp.where(jrow < total, out, out[:, 0:1])
    return out.astype(jnp.int32)


def _ball_query_pallas(radius, nsample, xyz, new_xyz, sqr=None):
    B, S, _ = new_xyz.shape
    N = xyz.shape[1]
    C = min(N, 128)
    K = N // C
    Sb = min(S, 256)
    const = jnp.asarray(_bq_consts(N, C, K))
    r2 = np.float32(radius * radius)
    if sqr is None:
        body = functools.partial(_bq_body, N=N, C=C, K=K, ns=nsample, r2=r2)
        xt = jnp.transpose(xyz, (0, 2, 1))
        specs = [
            pl.BlockSpec((1, Sb, 3), lambda b, s: (b, s, 0)),
            pl.BlockSpec((1, 3, N), lambda b, s: (b, 0, 0)),
        ]
        args = (new_xyz, xt)
    else:
        body = functools.partial(_bq_body_pre, N=N, C=C, K=K, ns=nsample, r2=r2)
        specs = [pl.BlockSpec((1, Sb, N), lambda b, s: (b, s, 0))]
        args = (sqr,)
    out = pl.pallas_call(
        body,
        grid=(B, S // Sb),
        in_specs=specs + [pl.BlockSpec(const.shape, lambda b, s: (0, 0))],
        out_specs=pl.BlockSpec((1, Sb, nsample), lambda b, s: (b, s, 0)),
        out_shape=jax.ShapeDtypeStruct((B, S, nsample), jnp.int32),
    )(*args, const)
    return out


def _knn3_body(x1_ref, x2t_ref, idx_ref, w_ref, *, M):
    x1 = x1_ref[0]
    x2t = x2t_ref[0]
    rn = jnp.sum(x1 * x1, axis=1, keepdims=True)
    rm = (x2t[0:1] * x2t[0:1] + x2t[1:2] * x2t[1:2]) + x2t[2:3] * x2t[2:3]
    ab = jnp.dot(x1, x2t, preferred_element_type=jnp.float32)
    d = (rn + rm) - 2.0 * ab
    _knn3_sel(d, idx_ref, w_ref, M)


def _knn3_body_pre(d_ref, idx_ref, w_ref, *, M):
    _knn3_sel(d_ref[0], idx_ref, w_ref, M)


def _knn3_sel(d, idx_ref, w_ref, M):
    iota = jax.lax.broadcasted_iota(jnp.int32, d.shape, 1)
    idxs, vals = [], []
    for _ in range(3):
        m = jnp.min(d, axis=1, keepdims=True)
        i = jnp.min(jnp.where(d == m, iota, M), axis=1, keepdims=True)
        d = jnp.where(iota == i, 1e30, d)
        idxs.append(i)
        vals.append(m)
    w = [1.0 / (jnp.maximum(v, 0.0) + 1e-8) for v in vals]
    ws = (w[0] + w[1]) + w[2]
    idx_ref[0] = jnp.concatenate(idxs, axis=1)
    w_ref[0] = jnp.concatenate([wj / ws for wj in w], axis=1)


def _knn3_pallas(xyz1, xyz2, d=None):
    B, S, _ = xyz1.shape
    M = xyz2.shape[1]
    Sb = min(S, 512)
    if d is None:
        body = functools.partial(_knn3_body, M=M)
        specs = [
            pl.BlockSpec((1, Sb, 3), lambda b, s: (b, s, 0)),
            pl.BlockSpec((1, 3, M), lambda b, s: (b, 0, 0)),
        ]
        args = (xyz1, jnp.transpose(xyz2, (0, 2, 1)))
    else:
        body = functools.partial(_knn3_body_pre, M=M)
        specs = [pl.BlockSpec((1, Sb, M), lambda b, s: (b, s, 0))]
        args = (d,)
    out = pl.pallas_call(
        body,
        grid=(B, S // Sb),
        in_specs=specs,
        out_specs=[
            pl.BlockSpec((1, Sb, 3), lambda b, s: (b, s, 0)),
            pl.BlockSpec((1, Sb, 3), lambda b, s: (b, s, 0)),
        ],
        out_shape=[
            jax.ShapeDtypeStruct((B, S, 3), jnp.int32),
            jax.ShapeDtypeStruct((B, S, 3), jnp.float32),
        ],
    )(*args)
    return out


def _sc_gather(table, idx):
    V, D = table.shape
    T = idx.shape[0]
    NW = 32
    rpw = T // NW
    chunk = min(rpw, max(8, 32768 // D))
    while rpw % chunk:
        chunk -= 8
    nch = rpw // chunk

    @functools.partial(
        pl.kernel,
        mesh=plsc.VectorSubcoreMesh(core_axis_name="c", subcore_axis_name="s"),
        out_type=jax.ShapeDtypeStruct((T, D), jnp.float32),
        scratch_types=[
            pltpu.VMEM((chunk,), jnp.int32),
            pltpu.VMEM((chunk, D), jnp.float32),
            pltpu.SemaphoreType.DMA,
        ],
    )
    def k(table_hbm, idx_hbm, out_hbm, idx_v, rows_v, sem):
        wid = lax.axis_index("s") * 2 + lax.axis_index("c")
        for j in range(nch):
            base = wid * rpw + j * chunk
            pltpu.sync_copy(idx_hbm.at[pl.ds(base, chunk)], idx_v)
            pltpu.async_copy(table_hbm.at[idx_v], rows_v, sem).wait()
            pltpu.sync_copy(rows_v, out_hbm.at[pl.ds(base, chunk)])

    return k(table, idx)


def _gather_rows(table, idx):
    B, V, D = table.shape
    T = idx.shape[1]
    rpw = (B * T) // 32
    ok = D % 128 == 0 and (B * T) % 256 == 0
    if ok:
        chunk = min(rpw, max(8, 32768 // D))
        while rpw % chunk:
            chunk -= 8
        ok = chunk >= 8 and chunk % 8 == 0
    if ok:
        flat_idx = (idx + jnp.arange(B, dtype=jnp.int32)[:, None] * V).reshape(-1)
        out = _sc_gather(table.reshape(B * V, D), flat_idx)
        return out.reshape(B, T, D)
    return _index_points(table, idx)

def _sa_table_body(xyz_ref, feat_ref, w1xt_ref, w1ft_ref, out_ref):
    a = jnp.dot(xyz_ref[0], w1xt_ref[...], preferred_element_type=jnp.float32)
    a = a + jnp.dot(feat_ref[0], w1ft_ref[...], preferred_element_type=jnp.float32)
    out_ref[0] = a


def _sa_table(xyz, feat, w1xt, w1ft):
    B, N, _ = xyz.shape
    C1 = w1xt.shape[1]
    return pl.pallas_call(
        _sa_table_body,
        grid=(B,),
        in_specs=[
            pl.BlockSpec((1, N, 3), lambda b: (b, 0, 0)),
            pl.BlockSpec((1, N, feat.shape[2]), lambda b: (b, 0, 0)),
            pl.BlockSpec(w1xt.shape, lambda b: (0, 0)),
            pl.BlockSpec(w1ft.shape, lambda b: (0, 0)),
        ],
        out_specs=pl.BlockSpec((1, N, C1), lambda b: (b, 0, 0)),
        out_shape=jax.ShapeDtypeStruct((B, N, C1), jnp.float32),
    )(xyz, feat, w1xt, w1ft)


def _sa_mlp_body(g_ref, nx_ref, w1xt_ref, b1_ref, w2t_ref, a2_ref, b2_ref,
                 w3t_ref, a3_ref, b3_ref, out_ref, *, Sb, ns):
    gth = g_ref[0]
    off = jnp.dot(nx_ref[0], w1xt_ref[...],
                  preferred_element_type=jnp.float32) - b1_ref[...]
    C1 = gth.shape[1]
    off3 = jnp.broadcast_to(off[:, None, :], (Sb, ns, C1)).reshape(Sb * ns, C1)
    x1 = jax.nn.relu(gth - off3)
    x2 = jnp.dot(x1, w2t_ref[...], preferred_element_type=jnp.float32)
    x2 = jax.nn.relu(x2 * a2_ref[...] + b2_ref[...])
    x3 = jnp.dot(x2, w3t_ref[...], preferred_element_type=jnp.float32)
    x3 = jax.nn.relu(x3 * a3_ref[...] + b3_ref[...])
    C3 = x3.shape[1]
    out_ref[0] = jnp.max(x3.reshape(Sb, ns, C3), axis=1)


def _sa_features(gathered, new_xyz, params, prefix, ns):
    B, S, _ = new_xyz.shape
    gs1 = params[prefix + '_g0'] * _BN_SCALE
    w1xt = (params[prefix + '_w0'][:, :3] * gs1[:, None]).T
    b1 = params[prefix + '_b0'][None, :]
    w2t = params[prefix + '_w1'].T
    a2 = (params[prefix + '_g1'] * _BN_SCALE)[None, :]
    b2 = params[prefix + '_b1'][None, :]
    w3t = params[prefix + '_w2'].T
    a3 = (params[prefix + '_g2'] * _BN_SCALE)[None, :]
    b3 = params[prefix + '_b2'][None, :]
    C1, C3 = w2t.shape[0], w3t.shape[1]
    Sb = min(S, 256)
    wspec = lambda a: pl.BlockSpec(a.shape, lambda b, s: tuple(0 for _ in a.shape))
    out = pl.pallas_call(
        functools.partial(_sa_mlp_body, Sb=Sb, ns=ns),
        grid=(B, S // Sb),
        in_specs=[
            pl.BlockSpec((1, Sb * ns, C1), lambda b, s: (b, s, 0)),
            pl.BlockSpec((1, Sb, 3), lambda b, s: (b, s, 0)),
            wspec(w1xt), wspec(b1), wspec(w2t), wspec(a2), wspec(b2),
            wspec(w3t), wspec(a3), wspec(b3),
        ],
        out_specs=pl.BlockSpec((1, Sb, C3), lambda b, s: (b, s, 0)),
        out_shape=jax.ShapeDtypeStruct((B, S, C3), jnp.float32),
    )(gathered, new_xyz, w1xt, b1, w2t, a2, b2, w3t, a3, b3)
    return out


def _fc_head_body(x_ref, w1_ref, a1_ref, b1_ref, w2_ref, b2_ref, out_ref):
    x = x_ref[...]
    h = jnp.dot(x, w1_ref[...].T, preferred_element_type=jnp.float32)
    h = jax.nn.relu(h * a1_ref[...] + b1_ref[...])
    out_ref[...] = jnp.dot(h, w2_ref[...].T, preferred_element_type=jnp.float32) + b2_ref[...]


def _fc_head(x, w1, g1, b1, w2, b2):
    R, C = x.shape
    w2p = jnp.zeros((128, C), jnp.float32).at[:_NUM_CLASSES].set(w2)
    b2p = jnp.zeros((1, 128), jnp.float32).at[0, :_NUM_CLASSES].set(b2)
    a1 = (g1 * _BN_SCALE)[None, :]
    blk = 2048
    out = pl.pallas_call(
        _fc_head_body,
        grid=(R // blk,),
        in_specs=[
            pl.BlockSpec((blk, C), lambda i: (i, 0)),
            pl.BlockSpec((C, C), lambda i: (0, 0)),
            pl.BlockSpec((1, C), lambda i: (0, 0)),
            pl.BlockSpec((1, C), lambda i: (0, 0)),
            pl.BlockSpec((C, C), lambda i: (0, 0)),
            pl.BlockSpec((1, C), lambda i: (0, 0)),
        ],
        out_specs=pl.BlockSpec((blk, 128), lambda i: (i, 0)),
        out_shape=jax.ShapeDtypeStruct((R, 128), jnp.float32),
    )(x, w1, a1, b1[None, :], w2p, b2p)
    return out[:, :_NUM_CLASSES]


def kernel(xyz, points, params):
    l1x, l1p = _sa_module(xyz, points, params, 'sa0', 1024, 0.1, 32)
    l2x, l2p = _sa_module(l1x, l1p, params, 'sa1', 256, 0.2, 32)
    l3x, l3p = _sa_module(l2x, l2p, params, 'sa2', 64, 0.4, 32)
    l4x, l4p = _sa_module(l3x, l3p, params, 'sa3', 16, 0.8, 32)
    l3p = _fp_module(l3x, l4x, l3p, l4p, params, 'fp0', 2)
    l2p = _fp_module(l2x, l3x, l2p, l3p, params, 'fp1', 2)
    l1p = _fp_module(l1x, l2x, l1p, l2p, params, 'fp2', 2)
    l0p = _fp_module(xyz, l1x, points, l1p, params, 'fp3', 3)
    B, N, C = l0p.shape
    out = _fc_head(l0p.reshape(B * N, C), params['fc1_w'], params['fc1_g'],
                   params['fc1_b'], params['fc2_w'], params['fc2_bias'])
    return out.reshape(B, N, _NUM_CLASSES)
